# baseline (speedup 1.0000x reference)
import functools

import jax
import jax.numpy as jnp
from jax import lax
from jax.experimental import pallas as pl
from jax.experimental.pallas import tpu as pltpu
from jax.experimental.pallas import tpu_sc as plsc


def _sc_gather(table, idx):
    V, D = table.shape
    (B,) = idx.shape
    info = plsc.get_sparse_core_info()
    nc = info.num_cores
    nw = nc * info.num_subcores
    b_per_w = B // nw
    lanes = info.num_lanes
    groups = b_per_w // lanes
    mesh = plsc.VectorSubcoreMesh(core_axis_name="c", subcore_axis_name="s")

    @functools.partial(
        pl.kernel,
        mesh=mesh,
        out_type=jax.ShapeDtypeStruct((B, D), jnp.float32),
        scratch_types=[
            pltpu.VMEM((b_per_w,), jnp.int32),
            pltpu.VMEM((b_per_w, D), jnp.float32),
            pltpu.SemaphoreType.DMA,
            pltpu.SemaphoreType.DMA,
            pltpu.SemaphoreType.DMA,
            pltpu.SemaphoreType.DMA,
            pltpu.SemaphoreType.DMA,
        ],
    )
    def k(table_hbm, idx_hbm, out_hbm, idx_v, rows_v, sem_i, s0, s1, s2, s3):
        sems = [s0, s1, s2, s3]
        wid = lax.axis_index("s") * nc + lax.axis_index("c")
        base = wid * b_per_w
        pltpu.async_copy(idx_hbm.at[pl.ds(base, b_per_w)], idx_v, sem_i).wait()

        half = b_per_w // 2

        def make_group_body(q):
            def group_body(g, _):
                vec = idx_v[pl.ds(g * lanes, lanes)]
                for l in range(lanes):
                    r = vec[l]
                    pltpu.async_copy(
                        table_hbm.at[pl.ds(r, 1), :],
                        rows_v.at[pl.ds(g * lanes + l, 1), :],
                        sems[q],
                    )
                return 0
            return group_body

        lax.fori_loop(0, groups // 2, make_group_body(0), 0)
        lax.fori_loop(groups // 2, groups, make_group_body(1), 0)
        for q in range(2):
            pltpu.make_async_copy(
                table_hbm.at[pl.ds(0, half), :],
                rows_v.at[pl.ds(q * half, half), :],
                sems[q],
            ).wait()
            pltpu.async_copy(
                rows_v.at[pl.ds(q * half, half), :],
                out_hbm.at[pl.ds(base + q * half, half)],
                sems[2 + q],
            )
        for q in range(2):
            pltpu.make_async_copy(
                rows_v.at[pl.ds(q * half, half), :],
                out_hbm.at[pl.ds(base + q * half, half)],
                sems[2 + q],
            ).wait()

    return k(table, idx)


def _tc_project(x, W, b):
    B, D = x.shape
    H = W.shape[1]
    blk = 2048

    def body(x_ref, w_ref, b_ref, o_ref):
        o_ref[...] = (
            jnp.dot(x_ref[...], w_ref[...], preferred_element_type=jnp.float32)
            + b_ref[...]
        )

    return pl.pallas_call(
        body,
        grid=(B // blk,),
        in_specs=[
            pl.BlockSpec((blk, D), lambda i: (i, 0)),
            pl.BlockSpec((D, H), lambda i: (0, 0)),
            pl.BlockSpec((1, H), lambda i: (0, 0)),
        ],
        out_specs=pl.BlockSpec((blk, H), lambda i: (i, 0)),
        out_shape=jax.ShapeDtypeStruct((B, H), jnp.float32),
    )(x, W, b.reshape(1, H))


def kernel(task_ids, table, W, b):
    rows = _sc_gather(table, task_ids.astype(jnp.int32))
    return _tc_project(rows, W, b)
